# Initial kernel scaffold; baseline (speedup 1.0000x reference)
import jax
import jax.numpy as jnp
from jax.experimental import pallas as pl


def kernel(x, edge_index, mask, params):
    raise NotImplementedError("write your pallas kernel here")


# speedup vs baseline: 8.4254x; 8.4254x over previous
import jax
import jax.numpy as jnp
from jax import lax
from jax.experimental import pallas as pl

_N = 50
_E = 800
_NEG = -1e30


def _rows_matmul(A, hT, n_out, n_in, a_is_out_in):
    rows = []
    for i in range(n_out):
        acc = None
        for k in range(n_in):
            w = A[i:i + 1, k:k + 1] if a_is_out_in else A[k:k + 1, i:i + 1]
            term = w * hT[k:k + 1, :]
            acc = term if acc is None else acc + term
        rows.append(acc)
    return jnp.concatenate(rows, axis=0)


def _gat_layer(hT, W, asrc, adst, bias, Sm, Dm, di, do):
    hWT = _rows_matmul(W, hT, do, di, a_is_out_in=False)
    s_srcT = None
    s_dstT = None
    for k in range(do):
        ts = asrc[k:k + 1, 0:1] * hWT[k:k + 1, :]
        td = adst[k:k + 1, 0:1] * hWT[k:k + 1, :]
        s_srcT = ts if s_srcT is None else s_srcT + ts
        s_dstT = td if s_dstT is None else s_dstT + td
    e_src = jnp.sum(Sm * s_srcT, axis=1, keepdims=True)
    e_dst = jnp.sum(Dm * s_dstT, axis=1, keepdims=True)
    e = e_src + e_dst
    e = jnp.where(e >= 0, e, 0.2 * e)
    e_loopT = s_srcT + s_dstT
    e_loopT = jnp.where(e_loopT >= 0, e_loopT, 0.2 * e_loopT)
    e_masked = jnp.where(Dm > 0, e, _NEG)
    e_maxT = jnp.maximum(jnp.max(e_masked, axis=0, keepdims=True), e_loopT)
    ed_max = jnp.sum(Dm * e_maxT, axis=1, keepdims=True)
    e_exp = jnp.exp(e - ed_max)
    e_exp_loopT = jnp.exp(e_loopT - e_maxT)
    denomT = jnp.sum(Dm * e_exp, axis=0, keepdims=True) + e_exp_loopT
    denom_e = jnp.sum(Dm * denomT, axis=1, keepdims=True)
    alpha = e_exp / (denom_e + 1e-16)
    alpha_loopT = e_exp_loopT / (denomT + 1e-16)
    rows = []
    for j in range(do):
        gj = jnp.sum(Sm * hWT[j:j + 1, :], axis=1, keepdims=True)
        scat = jnp.sum(Dm * (alpha * gj), axis=0, keepdims=True)
        rows.append(scat + alpha_loopT * hWT[j:j + 1, :] + bias[j:j + 1, 0:1])
    return jnp.concatenate(rows, axis=0)


def _pre_body(xT_ref, srcc_ref, dstc_ref, srcr_ref, dstr_ref,
              W0_ref, Ws_ref, asrcs_ref, adsts_ref, bs_ref,
              Wq_ref, Wk_ref, Wv_ref, Wo_ref,
              bq_ref, bk_ref, bv_ref, bo_ref,
              hmha_ref, asym_ref):
    srcc = srcc_ref[...]
    dstc = dstc_ref[...]
    iota_en = lax.broadcasted_iota(jnp.int32, (_E, _N), 1)
    Sm = (srcc == iota_en).astype(jnp.float32)
    Dm = (dstc == iota_en).astype(jnp.float32)
    iota_ne = lax.broadcasted_iota(jnp.int32, (_N, _E), 0)
    SmT = (srcr_ref[...] == iota_ne).astype(jnp.float32)
    DmT = (dstr_ref[...] == iota_ne).astype(jnp.float32)
    cnt = jnp.dot(SmT, Dm, preferred_element_type=jnp.float32)
    cntT = jnp.dot(DmT, Sm, preferred_element_type=jnp.float32)
    asym_ref[...] = ((cnt + cntT) > 0).astype(jnp.float32)

    hT = xT_ref[...]
    hT = _gat_layer(hT, W0_ref[...], asrcs_ref[0], adsts_ref[0], bs_ref[0],
                    Sm, Dm, 5, 3)
    hT = jnp.maximum(hT, 0.0)
    for i in range(1, 10):
        hT = _gat_layer(hT, Ws_ref[i - 1], asrcs_ref[i], adsts_ref[i],
                        bs_ref[i], Sm, Dm, 3, 3)
        if i < 9:
            hT = jnp.maximum(hT, 0.0)

    qT = _rows_matmul(Wq_ref[...], hT, 3, 3, a_is_out_in=True)
    kT = _rows_matmul(Wk_ref[...], hT, 3, 3, a_is_out_in=True)
    vT = _rows_matmul(Wv_ref[...], hT, 3, 3, a_is_out_in=True)
    bq = bq_ref[...]
    bk = bk_ref[...]
    bv = bv_ref[...]
    eye = (lax.broadcasted_iota(jnp.int32, (_N, _N), 0) ==
           lax.broadcasted_iota(jnp.int32, (_N, _N), 1)).astype(jnp.float32)
    ocols = []
    for i in range(3):
        qrow = qT[i:i + 1, :] + bq[i:i + 1, 0:1]
        krow = kT[i:i + 1, :] + bk[i:i + 1, 0:1]
        vrow = vT[i:i + 1, :] + bv[i:i + 1, 0:1]
        qcol = jnp.sum(eye * qrow, axis=1, keepdims=True)
        s = qcol * krow
        m = jnp.max(s, axis=1, keepdims=True)
        ex = jnp.exp(s - m)
        attn = ex / jnp.sum(ex, axis=1, keepdims=True)
        ocols.append(jnp.sum(attn * vrow, axis=1, keepdims=True))
    Wo = Wo_ref[...]
    bo = bo_ref[...]
    fcols = []
    for i in range(3):
        acc = None
        for j in range(3):
            term = ocols[j] * Wo[i:i + 1, j:j + 1]
            acc = term if acc is None else acc + term
        fcols.append(acc + bo[i:i + 1, 0:1])
    hmha_ref[...] = jnp.concatenate(fcols, axis=1)


def _head_body(h_ref, pe_ref, maskc_ref, A1_ref, b1_ref, A2_ref, b2_ref,
               A3_ref, b3_ref, cw_ref, cb_ref, res_ref, val_ref):
    x2 = jnp.concatenate([h_ref[...], pe_ref[...]], axis=1)
    hp = lax.Precision.HIGHEST
    r = jnp.dot(x2, A1_ref[...][0:52, :], precision=hp,
                preferred_element_type=jnp.float32) + b1_ref[...]
    r = jnp.maximum(r, 0.0)
    r = jnp.dot(r, A2_ref[...], precision=hp,
                preferred_element_type=jnp.float32) + b2_ref[...]
    r = jnp.maximum(r, 0.0)
    r = jnp.dot(r, A3_ref[...], precision=hp,
                preferred_element_type=jnp.float32) + b3_ref[...]
    res_ref[...] = r * maskc_ref[...]
    rc = r * cw_ref[0:1, 0:1] + cb_ref[0:1, 0:1]
    val_ref[...] = jnp.sum(rc, axis=0, keepdims=True) / float(_N)


def kernel(x, edge_index, mask, params):
    gat = params['gat']
    mha = params['mha']
    actor = params['actor']
    critic = params['critic']
    f32 = jnp.float32

    xT = x.T
    srcc = edge_index[0].reshape(_E, 1)
    dstc = edge_index[1].reshape(_E, 1)
    srcr = edge_index[0].reshape(1, _E)
    dstr = edge_index[1].reshape(1, _E)
    W0 = gat[0]['W']
    Ws = jnp.stack([gat[i]['W'] for i in range(1, 10)])
    asrcs = jnp.stack([g['a_src'].reshape(3, 1) for g in gat])
    adsts = jnp.stack([g['a_dst'].reshape(3, 1) for g in gat])
    bs = jnp.stack([g['b'].reshape(3, 1) for g in gat])

    hmha, asym = pl.pallas_call(
        _pre_body,
        out_shape=[jax.ShapeDtypeStruct((_N, 3), f32),
                   jax.ShapeDtypeStruct((_N, _N), f32)],
    )(xT, srcc, dstc, srcr, dstr,
      W0, Ws, asrcs, adsts, bs,
      mha['Wq'], mha['Wk'], mha['Wv'], mha['Wo'],
      mha['bq'].reshape(3, 1), mha['bk'].reshape(3, 1),
      mha['bv'].reshape(3, 1), mha['bo'].reshape(3, 1))

    deg = asym.sum(axis=1)
    dinv = jnp.where(deg > 0, 1.0 / jnp.sqrt(jnp.maximum(deg, 1e-12)), 0.0)
    Lm = jnp.eye(_N, dtype=f32) - (dinv[:, None] * asym) * dinv[None, :]
    _, evecs = jnp.linalg.eigh(Lm)
    pe = evecs[:, 1:_N]

    res, val = pl.pallas_call(
        _head_body,
        out_shape=[jax.ShapeDtypeStruct((_N, 1), f32),
                   jax.ShapeDtypeStruct((1, 1), f32)],
    )(hmha, pe, mask.reshape(_N, 1),
      actor['A1'], actor['b1'].reshape(1, 16),
      actor['A2'], actor['b2'].reshape(1, 32),
      actor['A3'], actor['b3'].reshape(1, 1),
      critic['cw'], critic['cb'].reshape(1, 1))

    return res.reshape(_N), val.reshape(())


# speedup vs baseline: 11.6369x; 1.3812x over previous
import jax
import jax.numpy as jnp
from jax import lax
from jax.experimental import pallas as pl

_N = 50
_E = 800
_NEG = -1e30


def _gat_layer(h, W, asrc, adst, bias, SmT, DmT, di, do):
    hW = []
    for j in range(do):
        acc = None
        for k in range(di):
            term = h[:, k:k + 1] * W[k:k + 1, j:j + 1]
            acc = term if acc is None else acc + term
        hW.append(acc)
    s_src = None
    s_dst = None
    for k in range(do):
        ts = hW[k] * asrc[0:1, k:k + 1]
        td = hW[k] * adst[0:1, k:k + 1]
        s_src = ts if s_src is None else s_src + ts
        s_dst = td if s_dst is None else s_dst + td
    e_row = (jnp.sum(SmT * s_src, axis=0, keepdims=True)
             + jnp.sum(DmT * s_dst, axis=0, keepdims=True))
    e_row = jnp.where(e_row >= 0, e_row, 0.2 * e_row)
    e_loop = s_src + s_dst
    e_loop = jnp.where(e_loop >= 0, e_loop, 0.2 * e_loop)
    e_max = jnp.maximum(
        jnp.max(jnp.where(DmT > 0, e_row, _NEG), axis=1, keepdims=True),
        e_loop)
    ed_max_row = jnp.sum(DmT * e_max, axis=0, keepdims=True)
    e_exp_row = jnp.exp(e_row - ed_max_row)
    e_exp_loop = jnp.exp(e_loop - e_max)
    denom = jnp.sum(DmT * e_exp_row, axis=1, keepdims=True) + e_exp_loop
    denom_row = jnp.sum(DmT * denom, axis=0, keepdims=True)
    alpha_row = e_exp_row / (denom_row + 1e-16)
    alpha_loop = e_exp_loop / (denom + 1e-16)
    cols = []
    for j in range(do):
        g_row = jnp.sum(SmT * hW[j], axis=0, keepdims=True)
        scat = jnp.sum(DmT * (alpha_row * g_row), axis=1, keepdims=True)
        cols.append(scat + alpha_loop * hW[j] + bias[0:1, j:j + 1])
    return jnp.concatenate(cols, axis=1)


def _pre_body(*refs):
    (x_ref, ei_ref, srcc_ref, dstc_ref) = refs[:4]
    gat_refs = refs[4:44]
    (Wq_ref, Wk_ref, Wv_ref, Wo_ref, bq_ref, bk_ref, bv_ref, bo_ref) = refs[44:52]
    hmha_ref, asym_ref = refs[52], refs[53]

    srcr = ei_ref[0:1, :]
    dstr = ei_ref[1:2, :]
    iota_ne = lax.broadcasted_iota(jnp.int32, (_N, _E), 0)
    SmT = (srcr == iota_ne).astype(jnp.float32)
    DmT = (dstr == iota_ne).astype(jnp.float32)
    iota_en = lax.broadcasted_iota(jnp.int32, (_E, _N), 1)
    Sm = (srcc_ref[...] == iota_en).astype(jnp.float32)
    Dm = (dstc_ref[...] == iota_en).astype(jnp.float32)
    cnt = jnp.dot(SmT, Dm, preferred_element_type=jnp.float32)
    cntT = jnp.dot(DmT, Sm, preferred_element_type=jnp.float32)
    asym_ref[...] = ((cnt + cntT) > 0).astype(jnp.float32)

    h = x_ref[...]
    dims = [(5, 3)] + [(3, 3)] * 9
    for i, (di, do) in enumerate(dims):
        W = gat_refs[4 * i][...]
        asrc = gat_refs[4 * i + 1][...].reshape(1, do)
        adst = gat_refs[4 * i + 2][...].reshape(1, do)
        bias = gat_refs[4 * i + 3][...].reshape(1, do)
        h = _gat_layer(h, W, asrc, adst, bias, SmT, DmT, di, do)
        if i < 9:
            h = jnp.maximum(h, 0.0)

    Wq, Wk, Wv, Wo = Wq_ref[...], Wk_ref[...], Wv_ref[...], Wo_ref[...]
    bq = bq_ref[...].reshape(1, 3)
    bk = bk_ref[...].reshape(1, 3)
    bv = bv_ref[...].reshape(1, 3)
    bo = bo_ref[...].reshape(1, 3)
    eye = (lax.broadcasted_iota(jnp.int32, (_N, _N), 0) ==
           lax.broadcasted_iota(jnp.int32, (_N, _N), 1)).astype(jnp.float32)

    def proj_cols(Wm, bm):
        cols = []
        for i in range(3):
            acc = None
            for j in range(3):
                term = h[:, j:j + 1] * Wm[i:i + 1, j:j + 1]
                acc = term if acc is None else acc + term
            cols.append(acc + bm[0:1, i:i + 1])
        return cols

    qcols = proj_cols(Wq, bq)
    kcols = proj_cols(Wk, bk)
    vcols = proj_cols(Wv, bv)
    ocols = []
    for i in range(3):
        krow = jnp.sum(eye * kcols[i], axis=0, keepdims=True)
        vrow = jnp.sum(eye * vcols[i], axis=0, keepdims=True)
        s = qcols[i] * krow
        m = jnp.max(s, axis=1, keepdims=True)
        ex = jnp.exp(s - m)
        attn = ex / jnp.sum(ex, axis=1, keepdims=True)
        ocols.append(jnp.sum(attn * vrow, axis=1, keepdims=True))
    fcols = []
    for i in range(3):
        acc = None
        for j in range(3):
            term = ocols[j] * Wo[i:i + 1, j:j + 1]
            acc = term if acc is None else acc + term
        fcols.append(acc + bo[0:1, i:i + 1])
    hmha_ref[...] = jnp.concatenate(fcols, axis=1)


def _head_body(h_ref, pe_ref, mask_ref, A1_ref, b1_ref, A2_ref, b2_ref,
               A3_ref, b3_ref, cw_ref, cb_ref, res_ref, val_ref):
    x2 = jnp.concatenate([h_ref[...], pe_ref[...]], axis=1)
    hp = lax.Precision.HIGHEST
    r = jnp.dot(x2, A1_ref[...][0:52, :], precision=hp,
                preferred_element_type=jnp.float32) + b1_ref[...].reshape(1, 16)
    r = jnp.maximum(r, 0.0)
    r = jnp.dot(r, A2_ref[...], precision=hp,
                preferred_element_type=jnp.float32) + b2_ref[...].reshape(1, 32)
    r = jnp.maximum(r, 0.0)
    r = jnp.dot(r, A3_ref[...], precision=hp,
                preferred_element_type=jnp.float32) + b3_ref[...].reshape(1, 1)
    eye = (lax.broadcasted_iota(jnp.int32, (_N, _N), 0) ==
           lax.broadcasted_iota(jnp.int32, (_N, _N), 1)).astype(jnp.float32)
    maskcol = jnp.sum(eye * mask_ref[...].reshape(1, _N), axis=1, keepdims=True)
    res_ref[...] = jnp.sum(eye * (r * maskcol), axis=0, keepdims=True)
    rc = r * cw_ref[0:1, 0:1] + cb_ref[...].reshape(1, 1)
    val_ref[...] = jnp.sum(rc, axis=0, keepdims=True) / float(_N)


def kernel(x, edge_index, mask, params):
    gat = params['gat']
    mha = params['mha']
    actor = params['actor']
    critic = params['critic']
    f32 = jnp.float32

    srcc = edge_index[0].reshape(_E, 1)
    dstc = edge_index[1].reshape(_E, 1)
    gat_args = []
    for g in gat:
        gat_args += [g['W'], g['a_src'], g['a_dst'], g['b']]

    hmha, asym = pl.pallas_call(
        _pre_body,
        out_shape=[jax.ShapeDtypeStruct((_N, 3), f32),
                   jax.ShapeDtypeStruct((_N, _N), f32)],
    )(x, edge_index, srcc, dstc, *gat_args,
      mha['Wq'], mha['Wk'], mha['Wv'], mha['Wo'],
      mha['bq'], mha['bk'], mha['bv'], mha['bo'])

    deg = asym.sum(axis=1)
    dinv = jnp.where(deg > 0, 1.0 / jnp.sqrt(jnp.maximum(deg, 1e-12)), 0.0)
    Lm = jnp.eye(_N, dtype=f32) - (dinv[:, None] * asym) * dinv[None, :]
    _, evecs = jnp.linalg.eigh(Lm)
    pe = evecs[:, 1:_N]

    res, val = pl.pallas_call(
        _head_body,
        out_shape=[jax.ShapeDtypeStruct((1, _N), f32),
                   jax.ShapeDtypeStruct((1, 1), f32)],
    )(hmha, pe, mask,
      actor['A1'], actor['b1'], actor['A2'], actor['b2'],
      actor['A3'], actor['b3'], critic['cw'], critic['cb'])

    return res.reshape(_N), val.reshape(())
